# baseline (speedup 1.0000x reference)
import jax
import jax.numpy as jnp
from jax import lax
from jax.experimental import pallas as pl
from jax.experimental.pallas import tpu as pltpu

_E = 16
_CAP = 1024
_MAX_FINITE_BITS = 0x7F7FFFFF


def _probs_body(h_ref, wt_ref, p_ref):
    x = h_ref[...]
    wt = wt_ref[...]
    logits = jnp.dot(x, wt, preferred_element_type=jnp.float32)
    logits = jnp.clip(logits, -10.0, 10.0)
    m = jnp.max(logits, axis=-1, keepdims=True)
    e = jnp.exp(logits - m)
    p_ref[...] = e / jnp.sum(e, axis=-1, keepdims=True)


def _count_ge(bits_ref, t):
    return jnp.sum((bits_ref[...] >= t).astype(jnp.int32), axis=0,
                   keepdims=True)


def _select_finalize_body(p_ref, w_ref, m_ref, bits_ref):
    n = p_ref.shape[0]
    bits_ref[...] = lax.bitcast_convert_type(p_ref[...], jnp.int32)

    def val_body(_, carry):
        lo, hi = carry
        step = jnp.maximum((hi - lo + 1) >> 3, 1)
        b = bits_ref[...]
        new_lo, new_hi = lo, hi
        for j in range(1, 8):
            m_j = jnp.minimum(lo + j * step, hi)
            cnt = jnp.sum((b >= m_j).astype(jnp.int32), axis=0, keepdims=True)
            ok = cnt >= _CAP
            new_lo = jnp.where(ok, jnp.maximum(new_lo, m_j), new_lo)
            new_hi = jnp.where(ok, new_hi, jnp.minimum(new_hi, m_j - 1))
        return new_lo, new_hi

    lo0 = jnp.zeros((1, _E), jnp.int32)
    hi0 = jnp.full((1, _E), _MAX_FINITE_BITS, jnp.int32)
    tbits, _ = lax.fori_loop(0, 12, val_body, (lo0, hi0))

    cgt = jnp.sum((bits_ref[...] > tbits).astype(jnp.int32), axis=0,
                  keepdims=True)
    need = _CAP - cgt

    idx = lax.broadcasted_iota(jnp.int32, (n, 1), 0)

    def idx_body(_, carry):
        lo, hi = carry
        step = jnp.maximum((hi - lo + 1) >> 3, 1)
        eq = bits_ref[...] == tbits
        new_lo, new_hi = lo, hi
        for j in range(1, 8):
            m_j = jnp.minimum(lo + j * step, hi)
            cnt = jnp.sum((eq & (idx <= m_j)).astype(jnp.int32), axis=0,
                          keepdims=True)
            ok = cnt >= need
            new_lo = jnp.where(ok, new_lo, jnp.maximum(new_lo, m_j + 1))
            new_hi = jnp.where(ok, jnp.minimum(new_hi, m_j), new_hi)
        return new_lo, new_hi

    ilo0 = jnp.zeros((1, _E), jnp.int32)
    ihi0 = jnp.full((1, _E), n - 1, jnp.int32)
    _, icut = lax.fori_loop(0, 6, idx_body, (ilo0, ihi0))

    p = p_ref[...]
    bits = bits_ref[...]
    mask = (bits > tbits) | ((bits == tbits) & (idx <= icut))
    maskf = mask.astype(jnp.float32)
    wun = maskf * p
    denom = jnp.sum(wun, axis=-1, keepdims=True) + 1e-10
    w_ref[...] = wun / denom
    m_ref[...] = maskf


def kernel(hidden_states, gate_weight):
    b, s, d = hidden_states.shape
    n = b * s
    h = hidden_states.reshape(n, d)
    wt = gate_weight.T

    tok_blk = 512
    probs = pl.pallas_call(
        _probs_body,
        grid=(n // tok_blk,),
        in_specs=[
            pl.BlockSpec((tok_blk, d), lambda i: (i, 0)),
            pl.BlockSpec((d, _E), lambda i: (0, 0)),
        ],
        out_specs=pl.BlockSpec((tok_blk, _E), lambda i: (i, 0)),
        out_shape=jax.ShapeDtypeStruct((n, _E), jnp.float32),
    )(h, wt)

    w, m = pl.pallas_call(
        _select_finalize_body,
        in_specs=[pl.BlockSpec((n, _E), lambda: (0, 0))],
        out_specs=[
            pl.BlockSpec((n, _E), lambda: (0, 0)),
            pl.BlockSpec((n, _E), lambda: (0, 0)),
        ],
        out_shape=[
            jax.ShapeDtypeStruct((n, _E), jnp.float32),
            jax.ShapeDtypeStruct((n, _E), jnp.float32),
        ],
        scratch_shapes=[pltpu.VMEM((n, _E), jnp.int32)],
    )(probs)

    return w.reshape(b, s, _E), m.reshape(b, s, _E)


# speedup vs baseline: 2.4322x; 2.4322x over previous
import jax
import jax.numpy as jnp
from jax import lax
from jax.experimental import pallas as pl
from jax.experimental.pallas import tpu as pltpu

_E = 16
_CAP = 1024
_MAX_FINITE_BITS = 0x7F7FFFFF
_L = 128


def _probs_body(h_ref, wt_ref, p_ref):
    x = h_ref[...]
    wt = wt_ref[...]
    logits = jnp.dot(x, wt, preferred_element_type=jnp.float32)
    logits = jnp.clip(logits, -10.0, 10.0)
    m = jnp.max(logits, axis=-1, keepdims=True)
    e = jnp.exp(logits - m)
    p_ref[...] = e / jnp.sum(e, axis=-1, keepdims=True)


def _select_finalize_body(p_ref, w_ref, m_ref, bits_ref, idx_ref):
    rows, lanes = p_ref.shape
    n = rows * lanes // _E
    bits_ref[...] = lax.bitcast_convert_type(p_ref[...], jnp.int32)

    li = lax.broadcasted_iota(jnp.int32, (lanes, lanes), 0)
    lj = lax.broadcasted_iota(jnp.int32, (lanes, lanes), 1)
    m_exp = ((li & (_E - 1)) == (lj & (_E - 1))).astype(jnp.float32)
    m_tok = ((li // _E) == (lj // _E)).astype(jnp.float32)

    idx_ref[...] = (lax.broadcasted_iota(jnp.int32, (rows, lanes), 0) *
                    (lanes // _E) +
                    (lax.broadcasted_iota(jnp.int32, (rows, lanes), 1) // _E))

    capf = float(_CAP)

    def count_exp(x_bool):
        s = jnp.sum(x_bool.astype(jnp.float32), axis=0, keepdims=True)
        return jnp.dot(s, m_exp, preferred_element_type=jnp.float32)

    def val_body(_, carry):
        lo, hi = carry
        mid = lo + ((hi - lo + 1) >> 1)
        ok = count_exp(bits_ref[...] >= mid) >= capf
        return jnp.where(ok, mid, lo), jnp.where(ok, hi, mid - 1)

    lo0 = jnp.zeros((1, lanes), jnp.int32)
    hi0 = jnp.full((1, lanes), _MAX_FINITE_BITS, jnp.int32)
    tbits, _ = lax.fori_loop(0, 31, val_body, (lo0, hi0))

    need = capf - count_exp(bits_ref[...] > tbits)

    def idx_body(_, carry):
        lo, hi = carry
        mid = (lo + hi) >> 1
        ok = count_exp((bits_ref[...] == tbits) &
                       (idx_ref[...] <= mid)) >= need
        return jnp.where(ok, lo, mid + 1), jnp.where(ok, mid, hi)

    ilo0 = jnp.zeros((1, lanes), jnp.int32)
    ihi0 = jnp.full((1, lanes), n - 1, jnp.int32)
    icut, _ = lax.fori_loop(0, 13, idx_body, (ilo0, ihi0))

    p = p_ref[...]
    bits = bits_ref[...]
    mask = (bits > tbits) | ((bits == tbits) & (idx_ref[...] <= icut))
    maskf = mask.astype(jnp.float32)
    wun = maskf * p
    denom = jnp.dot(wun, m_tok, preferred_element_type=jnp.float32) + 1e-10
    w_ref[...] = wun / denom
    m_ref[...] = maskf


def kernel(hidden_states, gate_weight):
    b, s, d = hidden_states.shape
    n = b * s
    h = hidden_states.reshape(n, d)
    wt = gate_weight.T

    tok_blk = 512
    probs = pl.pallas_call(
        _probs_body,
        grid=(n // tok_blk,),
        in_specs=[
            pl.BlockSpec((tok_blk, d), lambda i: (i, 0)),
            pl.BlockSpec((d, _E), lambda i: (0, 0)),
        ],
        out_specs=pl.BlockSpec((tok_blk, _E), lambda i: (i, 0)),
        out_shape=jax.ShapeDtypeStruct((n, _E), jnp.float32),
    )(h, wt)

    rows = n * _E // _L
    probs_packed = probs.reshape(rows, _L)

    w, m = pl.pallas_call(
        _select_finalize_body,
        in_specs=[pl.BlockSpec((rows, _L), lambda: (0, 0))],
        out_specs=[
            pl.BlockSpec((rows, _L), lambda: (0, 0)),
            pl.BlockSpec((rows, _L), lambda: (0, 0)),
        ],
        out_shape=[
            jax.ShapeDtypeStruct((rows, _L), jnp.float32),
            jax.ShapeDtypeStruct((rows, _L), jnp.float32),
        ],
        scratch_shapes=[
            pltpu.VMEM((rows, _L), jnp.int32),
            pltpu.VMEM((rows, _L), jnp.int32),
        ],
    )(probs_packed)

    return w.reshape(b, s, _E), m.reshape(b, s, _E)


# speedup vs baseline: 2.6045x; 1.0709x over previous
import jax
import jax.numpy as jnp
from jax import lax
from jax.experimental import pallas as pl
from jax.experimental.pallas import tpu as pltpu

_E = 16
_CAP = 1024
_MAX_FINITE_BITS = 0x7F7FFFFF
_L = 128


def _probs_body(h_ref, wt_ref, p_ref):
    x = h_ref[...]
    wt = wt_ref[...]
    logits = jnp.dot(x, wt, preferred_element_type=jnp.float32)
    logits = jnp.clip(logits, -10.0, 10.0)
    m = jnp.max(logits, axis=-1, keepdims=True)
    e = jnp.exp(logits - m)
    p_ref[...] = e / jnp.sum(e, axis=-1, keepdims=True)


def _select_finalize_body(p_ref, w_ref, m_ref, bits_ref, idx_ref, icut_ref):
    rows, lanes = p_ref.shape
    n = rows * lanes // _E
    bits_ref[...] = lax.bitcast_convert_type(p_ref[...], jnp.int32)

    li = lax.broadcasted_iota(jnp.int32, (lanes, lanes), 0)
    lj = lax.broadcasted_iota(jnp.int32, (lanes, lanes), 1)
    m_exp = ((li & (_E - 1)) == (lj & (_E - 1))).astype(jnp.float32)
    m_tok = ((li // _E) == (lj // _E)).astype(jnp.float32)

    idx_ref[...] = (lax.broadcasted_iota(jnp.int32, (rows, lanes), 0) *
                    (lanes // _E) +
                    (lax.broadcasted_iota(jnp.int32, (rows, lanes), 1) // _E))

    capf = float(_CAP)

    def count_exp(x_bool):
        s = jnp.sum(x_bool.astype(jnp.float32), axis=0, keepdims=True)
        return jnp.dot(s, m_exp, preferred_element_type=jnp.float32)

    def val_body(_, carry):
        lo, hi = carry
        mid = lo + ((hi - lo + 1) >> 1)
        ok = count_exp(bits_ref[...] >= mid) >= capf
        return jnp.where(ok, mid, lo), jnp.where(ok, hi, mid - 1)

    lo0 = jnp.full((1, lanes), 0x2F03F0FF, jnp.int32)
    hi0 = jnp.full((1, lanes), 0x3F800000, jnp.int32)
    tbits, _ = lax.fori_loop(0, 29, val_body, (lo0, hi0))

    icut_ref[...] = jnp.full((1, lanes), n - 1, jnp.int32)
    cge = count_exp(bits_ref[...] >= tbits)
    has_ties = jnp.any(cge > capf)

    @pl.when(has_ties)
    def _():
        need = capf - count_exp(bits_ref[...] > tbits)

        def idx_body(_, carry):
            lo, hi = carry
            mid = (lo + hi) >> 1
            ok = count_exp((bits_ref[...] == tbits) &
                           (idx_ref[...] <= mid)) >= need
            return jnp.where(ok, lo, mid + 1), jnp.where(ok, mid, hi)

        ilo0 = jnp.zeros((1, lanes), jnp.int32)
        ihi0 = jnp.full((1, lanes), n - 1, jnp.int32)
        res, _ = lax.fori_loop(0, 13, idx_body, (ilo0, ihi0))
        icut_ref[...] = res

    icut = icut_ref[...]

    p = p_ref[...]
    bits = bits_ref[...]
    mask = (bits > tbits) | ((bits == tbits) & (idx_ref[...] <= icut))
    maskf = mask.astype(jnp.float32)
    wun = maskf * p
    denom = jnp.dot(wun, m_tok, preferred_element_type=jnp.float32) + 1e-10
    w_ref[...] = wun / denom
    m_ref[...] = maskf


def kernel(hidden_states, gate_weight):
    b, s, d = hidden_states.shape
    n = b * s
    h = hidden_states.reshape(n, d)
    wt = gate_weight.T

    tok_blk = 512
    probs = pl.pallas_call(
        _probs_body,
        grid=(n // tok_blk,),
        in_specs=[
            pl.BlockSpec((tok_blk, d), lambda i: (i, 0)),
            pl.BlockSpec((d, _E), lambda i: (0, 0)),
        ],
        out_specs=pl.BlockSpec((tok_blk, _E), lambda i: (i, 0)),
        out_shape=jax.ShapeDtypeStruct((n, _E), jnp.float32),
    )(h, wt)

    rows = n * _E // _L
    probs_packed = probs.reshape(rows, _L)

    w, m = pl.pallas_call(
        _select_finalize_body,
        in_specs=[pl.BlockSpec((rows, _L), lambda: (0, 0))],
        out_specs=[
            pl.BlockSpec((rows, _L), lambda: (0, 0)),
            pl.BlockSpec((rows, _L), lambda: (0, 0)),
        ],
        out_shape=[
            jax.ShapeDtypeStruct((rows, _L), jnp.float32),
            jax.ShapeDtypeStruct((rows, _L), jnp.float32),
        ],
        scratch_shapes=[
            pltpu.VMEM((rows, _L), jnp.int32),
            pltpu.VMEM((rows, _L), jnp.int32),
            pltpu.VMEM((1, _L), jnp.int32),
        ],
    )(probs_packed)

    return w.reshape(b, s, _E), m.reshape(b, s, _E)
